# baseline (speedup 1.0000x reference)
import functools

import jax
import jax.numpy as jnp
from jax import lax
from jax.experimental import pallas as pl
from jax.experimental.pallas import tpu as pltpu
from jax.experimental.pallas import tpu_sc as plsc

VOCAB = 1_000_000
EMBED_DIM = 128
N_CLASSES = 5
BATCH = 16384
HIST = 200
PROJ_DIM = 16

_ROWS_BLK = 1000
_VCHUNK = VOCAB // 8


def _proj_body(x0, x1, x2, x3, x4, x5, x6, x7, w_ref, o_ref):
    x = jnp.concatenate(
        [x0[...], x1[...], x2[...], x3[...], x4[...], x5[...], x6[...],
         x7[...]], axis=1)
    o_ref[...] = jnp.dot(x, w_ref[...], preferred_element_type=jnp.float32)


def _project(table, w8):
    nblk = _VCHUNK // _ROWS_BLK
    in_specs = [
        pl.BlockSpec((_ROWS_BLK, EMBED_DIM),
                     lambda i, p=p: (p * nblk + i, 0))
        for p in range(8)
    ] + [pl.BlockSpec((8 * EMBED_DIM, 8 * PROJ_DIM), lambda i: (0, 0))]
    return pl.pallas_call(
        _proj_body,
        grid=(nblk,),
        in_specs=in_specs,
        out_specs=pl.BlockSpec((_ROWS_BLK, 8 * PROJ_DIM), lambda i: (i, 0)),
        out_shape=jax.ShapeDtypeStruct((_VCHUNK, 8 * PROJ_DIM), jnp.float32),
    )(*([table] * 8), w8)


_NC = 2
_NS = 16
_NW = _NC * _NS
_S_PER_W = BATCH // _NW
_IDX_PER_W = _S_PER_W * HIST
_CHUNK0 = 80
_CHUNK1 = HIST - _CHUNK0


@functools.partial(
    pl.kernel,
    out_type=jax.ShapeDtypeStruct((BATCH, PROJ_DIM), jnp.float32),
    mesh=plsc.VectorSubcoreMesh(
        core_axis_name="c", subcore_axis_name="s",
        num_cores=_NC, num_subcores=_NS),
    compiler_params=pltpu.CompilerParams(use_tc_tiling_on_sc=False),
    scratch_types=[
        pltpu.VMEM((_IDX_PER_W,), jnp.int32),
        pltpu.VMEM((2 * HIST, PROJ_DIM), jnp.float32),
        pltpu.VMEM((2 * HIST, PROJ_DIM), jnp.float32),
        pltpu.VMEM((2 * HIST, PROJ_DIM), jnp.float32),
        pltpu.VMEM((2 * HIST, PROJ_DIM), jnp.float32),
        pltpu.VMEM((2, PROJ_DIM), jnp.float32),
        pltpu.VMEM((2, PROJ_DIM), jnp.float32),
        pltpu.VMEM((2, PROJ_DIM), jnp.float32),
        pltpu.VMEM((2, PROJ_DIM), jnp.float32),
        pltpu.VMEM((PROJ_DIM,), jnp.float32),
        pltpu.SemaphoreType.DMA,
        pltpu.SemaphoreType.DMA,
        pltpu.SemaphoreType.DMA,
        pltpu.SemaphoreType.DMA,
        pltpu.SemaphoreType.DMA,
    ],
)
def _pool(proj_hbm, idx_hbm, bpad_hbm, out_hbm, idx_v, buf0_v, buf1_v, buf2_v,
          buf3_v, ob0_v, ob1_v, ob2_v, ob3_v, b_v, gsem0, gsem1, gsem2, gsem3,
          osem):
    bufs = (buf0_v, buf1_v, buf2_v, buf3_v)
    obufs = (ob0_v, ob1_v, ob2_v, ob3_v)
    gsems = (gsem0, gsem1, gsem2, gsem3)
    wid = lax.axis_index("s") * _NC + lax.axis_index("c")
    out_base = wid * _S_PER_W
    pltpu.sync_copy(idx_hbm.at[pl.ds(wid * _IDX_PER_W, _IDX_PER_W)], idx_v)
    pltpu.sync_copy(bpad_hbm, b_v)
    bvec = b_v[...]

    _inv = jnp.float32((1.0 + 2.0 ** -20) / _VCHUNK)

    def xform_body(t, carry):
        base = pl.multiple_of(t * 128, 128)
        for u in range(8):
            off = base + u * 16
            v = idx_v[pl.ds(off, 16)]
            q = (v.astype(jnp.float32) * _inv).astype(jnp.int32)
            r = v - q * jnp.int32(_VCHUNK)
            idx_v[pl.ds(off, 16)] = r * 8 + q
        return carry

    lax.fori_loop(0, _IDX_PER_W // 128, xform_body, 0)

    npairs = _S_PER_W // 2

    def fire(p, j):
        off = pl.multiple_of(p * 2 * HIST, 8)
        pltpu.async_copy(proj_hbm.at[idx_v.at[pl.ds(off, 2 * HIST)]],
                         bufs[j], gsems[j])

    def reduce_sample(buf, base_row):
        def red_body(t, accs):
            a0, a1, a2, a3 = accs
            r = base_row + t * 8
            a0 = a0 + buf[r]
            a1 = a1 + buf[r + 1]
            a2 = a2 + buf[r + 2]
            a3 = a3 + buf[r + 3]
            a0 = a0 + buf[r + 4]
            a1 = a1 + buf[r + 5]
            a2 = a2 + buf[r + 6]
            a3 = a3 + buf[r + 7]
            return (a0, a1, a2, a3)

        zero = jnp.zeros((PROJ_DIM,), jnp.float32)
        a0, a1, a2, a3 = lax.fori_loop(0, HIST // 8, red_body,
                                       (bvec, zero, zero, zero))
        return (a0 + a1) + (a2 + a3)

    for j in range(3):
        fire(j, j)

    def body(g, carry):
        for j in range(4):
            p = 4 * g + j
            pltpu.make_async_copy(proj_hbm.at[pl.ds(0, 2 * HIST)], bufs[j],
                                  gsems[j]).wait()
            acc_a = reduce_sample(bufs[j], 0)
            acc_b = reduce_sample(bufs[j], HIST)

            @pl.when(p >= 4)
            def _():
                pltpu.make_async_copy(proj_hbm.at[pl.ds(0, 2)], obufs[j],
                                      osem).wait()

            obufs[j][0] = acc_a
            obufs[j][1] = acc_b
            pltpu.async_copy(obufs[j], out_hbm.at[pl.ds(out_base + 2 * p, 2)],
                             osem)

            @pl.when(p + 3 < npairs)
            def _():
                fire(p + 3, (j + 3) % 4)
        return carry

    lax.fori_loop(0, npairs // 4, body, 0)
    for j in range(4):
        pltpu.make_async_copy(proj_hbm.at[pl.ds(0, 2)], obufs[j], osem).wait()


def kernel(inputs, embed_table, W, b):
    w_pad = jnp.zeros((EMBED_DIM, PROJ_DIM), jnp.float32).at[:, :N_CLASSES].set(W)
    w8 = jax.scipy.linalg.block_diag(*([w_pad] * 8))
    b_pad = jnp.zeros((PROJ_DIM,), jnp.float32).at[:N_CLASSES].set(b)
    proj = _project(embed_table, w8).reshape(VOCAB, PROJ_DIM)
    pooled = _pool(proj, inputs.reshape(-1), b_pad)
    return pooled[:, :N_CLASSES]


# speedup vs baseline: 1.0855x; 1.0855x over previous
import functools

import jax
import jax.numpy as jnp
from jax import lax
from jax.experimental import pallas as pl
from jax.experimental.pallas import tpu as pltpu
from jax.experimental.pallas import tpu_sc as plsc

VOCAB = 1_000_000
EMBED_DIM = 128
N_CLASSES = 5
BATCH = 16384
HIST = 200
PROJ_DIM = 16

_ROWS_BLK = 5000
_VCHUNK = VOCAB // 8


def _proj_body(x0, x1, x2, x3, x4, x5, x6, x7, w_ref, o_ref):
    x = jnp.concatenate(
        [x0[...], x1[...], x2[...], x3[...], x4[...], x5[...], x6[...],
         x7[...]], axis=1)
    o_ref[...] = jnp.dot(x, w_ref[...], preferred_element_type=jnp.float32)


def _project(table, w8):
    nblk = _VCHUNK // _ROWS_BLK
    in_specs = [
        pl.BlockSpec((_ROWS_BLK, EMBED_DIM),
                     lambda i, p=p: (p * nblk + i, 0))
        for p in range(8)
    ] + [pl.BlockSpec((8 * EMBED_DIM, 8 * PROJ_DIM), lambda i: (0, 0))]
    return pl.pallas_call(
        _proj_body,
        grid=(nblk,),
        in_specs=in_specs,
        out_specs=pl.BlockSpec((_ROWS_BLK, 8 * PROJ_DIM), lambda i: (i, 0)),
        out_shape=jax.ShapeDtypeStruct((_VCHUNK, 8 * PROJ_DIM), jnp.float32),
    )(*([table] * 8), w8)


_NC = 2
_NS = 16
_NW = _NC * _NS
_S_PER_W = BATCH // _NW
_IDX_PER_W = _S_PER_W * HIST
_CHUNK0 = 80
_CHUNK1 = HIST - _CHUNK0


@functools.partial(
    pl.kernel,
    out_type=jax.ShapeDtypeStruct((BATCH, PROJ_DIM), jnp.float32),
    mesh=plsc.VectorSubcoreMesh(
        core_axis_name="c", subcore_axis_name="s",
        num_cores=_NC, num_subcores=_NS),
    compiler_params=pltpu.CompilerParams(use_tc_tiling_on_sc=False),
    scratch_types=[
        pltpu.VMEM((_IDX_PER_W,), jnp.int32),
        pltpu.VMEM((2 * HIST, PROJ_DIM), jnp.float32),
        pltpu.VMEM((2 * HIST, PROJ_DIM), jnp.float32),
        pltpu.VMEM((2 * HIST, PROJ_DIM), jnp.float32),
        pltpu.VMEM((2 * HIST, PROJ_DIM), jnp.float32),
        pltpu.VMEM((2, PROJ_DIM), jnp.float32),
        pltpu.VMEM((2, PROJ_DIM), jnp.float32),
        pltpu.VMEM((2, PROJ_DIM), jnp.float32),
        pltpu.VMEM((2, PROJ_DIM), jnp.float32),
        pltpu.VMEM((PROJ_DIM,), jnp.float32),
        pltpu.SemaphoreType.DMA,
        pltpu.SemaphoreType.DMA,
        pltpu.SemaphoreType.DMA,
        pltpu.SemaphoreType.DMA,
        pltpu.SemaphoreType.DMA,
    ],
)
def _pool(proj_hbm, idx_hbm, bpad_hbm, out_hbm, idx_v, buf0_v, buf1_v, buf2_v,
          buf3_v, ob0_v, ob1_v, ob2_v, ob3_v, b_v, gsem0, gsem1, gsem2, gsem3,
          osem):
    bufs = (buf0_v, buf1_v, buf2_v, buf3_v)
    obufs = (ob0_v, ob1_v, ob2_v, ob3_v)
    gsems = (gsem0, gsem1, gsem2, gsem3)
    wid = lax.axis_index("s") * _NC + lax.axis_index("c")
    out_base = wid * _S_PER_W
    pltpu.sync_copy(idx_hbm.at[pl.ds(wid * _IDX_PER_W, _IDX_PER_W)], idx_v)
    pltpu.sync_copy(bpad_hbm, b_v)
    bvec = b_v[...]

    _inv = jnp.float32((1.0 + 2.0 ** -20) / _VCHUNK)

    def xform_body(t, carry):
        base = pl.multiple_of(t * 128, 128)
        for u in range(8):
            off = base + u * 16
            v = idx_v[pl.ds(off, 16)]
            q = (v.astype(jnp.float32) * _inv).astype(jnp.int32)
            r = v - q * jnp.int32(_VCHUNK)
            idx_v[pl.ds(off, 16)] = r * 8 + q
        return carry

    lax.fori_loop(0, _IDX_PER_W // 128, xform_body, 0)

    npairs = _S_PER_W // 2

    def fire(p, j):
        off = pl.multiple_of(p * 2 * HIST, 8)
        pltpu.async_copy(proj_hbm.at[idx_v.at[pl.ds(off, 2 * HIST)]],
                         bufs[j], gsems[j])

    def reduce_sample(buf, base_row):
        def red_body(t, accs):
            a0, a1, a2, a3 = accs
            r = base_row + t * 8
            a0 = a0 + buf[r]
            a1 = a1 + buf[r + 1]
            a2 = a2 + buf[r + 2]
            a3 = a3 + buf[r + 3]
            a0 = a0 + buf[r + 4]
            a1 = a1 + buf[r + 5]
            a2 = a2 + buf[r + 6]
            a3 = a3 + buf[r + 7]
            return (a0, a1, a2, a3)

        zero = jnp.zeros((PROJ_DIM,), jnp.float32)
        a0, a1, a2, a3 = lax.fori_loop(0, HIST // 8, red_body,
                                       (bvec, zero, zero, zero))
        return (a0 + a1) + (a2 + a3)

    for j in range(3):
        fire(j, j)

    def body(g, carry):
        for j in range(4):
            p = 4 * g + j
            pltpu.make_async_copy(proj_hbm.at[pl.ds(0, 2 * HIST)], bufs[j],
                                  gsems[j]).wait()
            acc_a = reduce_sample(bufs[j], 0)
            acc_b = reduce_sample(bufs[j], HIST)

            @pl.when(p >= 4)
            def _():
                pltpu.make_async_copy(proj_hbm.at[pl.ds(0, 2)], obufs[j],
                                      osem).wait()

            obufs[j][0] = acc_a
            obufs[j][1] = acc_b
            pltpu.async_copy(obufs[j], out_hbm.at[pl.ds(out_base + 2 * p, 2)],
                             osem)

            @pl.when(p + 3 < npairs)
            def _():
                fire(p + 3, (j + 3) % 4)
        return carry

    lax.fori_loop(0, npairs // 4, body, 0)
    for j in range(4):
        pltpu.make_async_copy(proj_hbm.at[pl.ds(0, 2)], obufs[j], osem).wait()


def kernel(inputs, embed_table, W, b):
    w_pad = jnp.zeros((EMBED_DIM, PROJ_DIM), jnp.float32).at[:, :N_CLASSES].set(W)
    w8 = jax.scipy.linalg.block_diag(*([w_pad] * 8))
    b_pad = jnp.zeros((PROJ_DIM,), jnp.float32).at[:N_CLASSES].set(b)
    proj = _project(embed_table, w8).reshape(VOCAB, PROJ_DIM)
    pooled = _pool(proj, inputs.reshape(-1), b_pad)
    return pooled[:, :N_CLASSES]


# speedup vs baseline: 1.1130x; 1.0253x over previous
import functools

import jax
import jax.numpy as jnp
from jax import lax
from jax.experimental import pallas as pl
from jax.experimental.pallas import tpu as pltpu
from jax.experimental.pallas import tpu_sc as plsc

VOCAB = 1_000_000
EMBED_DIM = 128
N_CLASSES = 5
BATCH = 16384
HIST = 200
PROJ_DIM = 16

_ROWS_BLK = 5000
_VCHUNK = VOCAB // 8


def _proj_body(x0, x1, x2, x3, x4, x5, x6, x7, w_ref, o_ref):
    x = jnp.concatenate(
        [x0[...], x1[...], x2[...], x3[...], x4[...], x5[...], x6[...],
         x7[...]], axis=1)
    o_ref[...] = jnp.dot(x, w_ref[...], preferred_element_type=jnp.float32)


def _project(table, w8):
    nblk = _VCHUNK // _ROWS_BLK
    in_specs = [
        pl.BlockSpec((_ROWS_BLK, EMBED_DIM),
                     lambda i, p=p: (p * nblk + i, 0))
        for p in range(8)
    ] + [pl.BlockSpec((8 * EMBED_DIM, 8 * PROJ_DIM), lambda i: (0, 0))]
    return pl.pallas_call(
        _proj_body,
        grid=(nblk,),
        in_specs=in_specs,
        out_specs=pl.BlockSpec((_ROWS_BLK, 8 * PROJ_DIM), lambda i: (i, 0)),
        out_shape=jax.ShapeDtypeStruct((_VCHUNK, 8 * PROJ_DIM), jnp.float32),
    )(*([table] * 8), w8)


_NC = 2
_NS = 16
_NW = _NC * _NS
_S_PER_W = BATCH // _NW
_IDX_PER_W = _S_PER_W * HIST
_CHUNK0 = 80
_CHUNK1 = HIST - _CHUNK0


@functools.partial(
    pl.kernel,
    out_type=jax.ShapeDtypeStruct((BATCH, PROJ_DIM), jnp.float32),
    mesh=plsc.VectorSubcoreMesh(
        core_axis_name="c", subcore_axis_name="s",
        num_cores=_NC, num_subcores=_NS),
    compiler_params=pltpu.CompilerParams(use_tc_tiling_on_sc=False),
    scratch_types=[
        pltpu.VMEM((_IDX_PER_W // 2,), jnp.int32),
        pltpu.VMEM((2 * HIST, PROJ_DIM), jnp.float32),
        pltpu.VMEM((2 * HIST, PROJ_DIM), jnp.float32),
        pltpu.VMEM((2 * HIST, PROJ_DIM), jnp.float32),
        pltpu.VMEM((2 * HIST, PROJ_DIM), jnp.float32),
        pltpu.VMEM((2 * HIST, PROJ_DIM), jnp.float32),
        pltpu.VMEM((2 * HIST, PROJ_DIM), jnp.float32),
        pltpu.VMEM((2 * HIST, PROJ_DIM), jnp.float32),
        pltpu.VMEM((2 * HIST, PROJ_DIM), jnp.float32),
        pltpu.VMEM((8, 2, PROJ_DIM), jnp.float32),
        pltpu.VMEM((PROJ_DIM,), jnp.float32),
        pltpu.SemaphoreType.DMA,
        pltpu.SemaphoreType.DMA,
        pltpu.SemaphoreType.DMA,
        pltpu.SemaphoreType.DMA,
        pltpu.SemaphoreType.DMA,
        pltpu.SemaphoreType.DMA,
        pltpu.SemaphoreType.DMA,
        pltpu.SemaphoreType.DMA,
        pltpu.SemaphoreType.DMA,
    ],
)
def _pool(proj_hbm, idx_hbm, bpad_hbm, out_hbm, idx_v, buf0_v, buf1_v, buf2_v,
          buf3_v, buf4_v, buf5_v, buf6_v, buf7_v, ob_v, b_v, gsem0, gsem1,
          gsem2, gsem3, gsem4, gsem5, gsem6, gsem7, osem):
    bufs = (buf0_v, buf1_v, buf2_v, buf3_v, buf4_v, buf5_v, buf6_v, buf7_v)
    gsems = (gsem0, gsem1, gsem2, gsem3, gsem4, gsem5, gsem6, gsem7)
    wid = lax.axis_index("s") * _NC + lax.axis_index("c")
    out_base = wid * _S_PER_W
    pltpu.sync_copy(bpad_hbm, b_v)
    bvec = b_v[...]

    nhalf = _IDX_PER_W // 2
    hpairs = _S_PER_W // 4

    _inv = jnp.float32((1.0 + 2.0 ** -20) / _VCHUNK)

    def xform_body(t, carry):
        base = pl.multiple_of(t * 128, 128)
        for u in range(8):
            off = base + u * 16
            v = idx_v[pl.ds(off, 16)]
            q = (v.astype(jnp.float32) * _inv).astype(jnp.int32)
            r = v - q * jnp.int32(_VCHUNK)
            idx_v[pl.ds(off, 16)] = r * 8 + q
        return carry

    def fire(p, j):
        off = pl.multiple_of(p * 2 * HIST, 8)
        pltpu.async_copy(proj_hbm.at[idx_v.at[pl.ds(off, 2 * HIST)]],
                         bufs[j], gsems[j])

    def reduce_sample(buf, base_row):
        def red_body(t, accs):
            a0, a1, a2, a3 = accs
            r = base_row + t * 8
            a0 = a0 + buf[r]
            a1 = a1 + buf[r + 1]
            a2 = a2 + buf[r + 2]
            a3 = a3 + buf[r + 3]
            a0 = a0 + buf[r + 4]
            a1 = a1 + buf[r + 5]
            a2 = a2 + buf[r + 6]
            a3 = a3 + buf[r + 7]
            return (a0, a1, a2, a3)

        zero = jnp.zeros((PROJ_DIM,), jnp.float32)
        a0, a1, a2, a3 = lax.fori_loop(0, HIST // 8, red_body,
                                       (bvec, zero, zero, zero))
        return (a0 + a1) + (a2 + a3)

    for h in range(2):
        pltpu.sync_copy(
            idx_hbm.at[pl.ds(wid * _IDX_PER_W + h * nhalf, nhalf)], idx_v)
        lax.fori_loop(0, nhalf // 128, xform_body, 0)
        for j in range(7):
            fire(j, j)

        def body(g, carry):
            for j in range(8):
                p = 8 * g + j
                pltpu.make_async_copy(proj_hbm.at[pl.ds(0, 2 * HIST)],
                                      bufs[j], gsems[j]).wait()
                acc_a = reduce_sample(bufs[j], 0)
                acc_b = reduce_sample(bufs[j], HIST)

                @pl.when(h * hpairs + p >= 8)
                def _():
                    pltpu.make_async_copy(proj_hbm.at[pl.ds(0, 2)],
                                          ob_v.at[j], osem).wait()

                ob_v[j, 0] = acc_a
                ob_v[j, 1] = acc_b
                pltpu.async_copy(
                    ob_v.at[j],
                    out_hbm.at[pl.ds(out_base + h * 2 * hpairs + 2 * p, 2)],
                    osem)

                @pl.when(p + 7 < hpairs)
                def _():
                    fire(p + 7, (j + 7) % 8)
            return carry

        lax.fori_loop(0, hpairs // 8, body, 0)
    for j in range(8):
        pltpu.make_async_copy(proj_hbm.at[pl.ds(0, 2)], ob_v.at[j],
                              osem).wait()


def kernel(inputs, embed_table, W, b):
    w_pad = jnp.zeros((EMBED_DIM, PROJ_DIM), jnp.float32).at[:, :N_CLASSES].set(W)
    w8 = jax.scipy.linalg.block_diag(*([w_pad] * 8))
    b_pad = jnp.zeros((PROJ_DIM,), jnp.float32).at[:N_CLASSES].set(b)
    proj = _project(embed_table, w8).reshape(VOCAB, PROJ_DIM)
    pooled = _pool(proj, inputs.reshape(-1), b_pad)
    return pooled[:, :N_CLASSES]
